# Initial kernel scaffold; baseline (speedup 1.0000x reference)
import jax
import jax.numpy as jnp
from jax.experimental import pallas as pl


def kernel(input_data, emb_table, W, b):
    raise NotImplementedError("write your pallas kernel here")


# speedup vs baseline: 5.9036x; 5.9036x over previous
import jax
import jax.numpy as jnp
from jax import lax
from jax.experimental import pallas as pl
from jax.experimental.pallas import tpu as pltpu
from jax.experimental.pallas import tpu_sc as plsc

V = 100000
D = 64
L = 200
B = 1024
NCLS = 2
CH = 100
NW = 32
SPW = B // NW
G = 2
NG = SPW // G
CPG = G * L // CH
CPW = SPW * L // CH


def _sc_body(idx_hbm, tab_hbm, wt_hbm, b_hbm, out_hbm,
             idx_v, w_v, b_v, rows0, rows1, lg0_v, lg1_v, o_v, sem0, sem1):
    cid = lax.axis_index("c")
    sid = lax.axis_index("s")
    wid = sid * 2 + cid

    pltpu.sync_copy(idx_hbm.at[pl.ds(wid * CPW, CPW)], idx_v)
    pltpu.sync_copy(wt_hbm, w_v)
    pltpu.sync_copy(b_hbm, b_v)

    bufs = (rows0, rows1)
    sems = (sem0, sem1)

    def start_group(g, p):
        for k in range(CPG):
            pltpu.async_copy(tab_hbm.at[idx_v.at[g * CPG + k]],
                             bufs[p].at[pl.ds(k * CH, CH)], sems[p])

    def wait_group(p):
        for k in range(CPG):
            pltpu.make_async_copy(tab_hbm.at[idx_v.at[k]],
                                  bufs[p].at[pl.ds(k * CH, CH)],
                                  sems[p]).wait()

    def compute_group(g, p):
        rows = bufs[p]
        for s in range(G):
            def jbody(j, accs):
                r = s * L + j
                f = [rows[r, pl.ds(d * 16, 16)] for d in range(4)]
                off = j * D
                new = []
                for c in range(NCLS):
                    for d in range(4):
                        wv = w_v[c, pl.ds(off + d * 16, 16)]
                        new.append(accs[c * 4 + d] + f[d] * wv)
                return tuple(new)

            z16 = jnp.zeros((16,), jnp.float32)
            accs = lax.fori_loop(0, L, jbody, (z16,) * 8)
            sg = g * G + s
            lane0 = lax.iota(jnp.int32, 16) == 0
            idxv = jnp.full((16,), sg, dtype=jnp.int32)
            s0 = jnp.full((16,), jnp.sum(accs[0] + accs[1] + accs[2] + accs[3]))
            s1 = jnp.full((16,), jnp.sum(accs[4] + accs[5] + accs[6] + accs[7]))
            plsc.store_scatter(lg0_v, [idxv], s0, mask=lane0)
            plsc.store_scatter(lg1_v, [idxv], s1, mask=lane0)

    start_group(0, 0)

    @pl.loop(0, NG, step=2)
    def _(g):
        start_group(g + 1, 1)
        wait_group(0)
        compute_group(g, 0)

        @pl.when(g + 2 < NG)
        def _():
            start_group(g + 2, 0)

        wait_group(1)
        compute_group(g + 1, 1)

    for t in range(SPW // 16):
        v0 = lg0_v[pl.ds(t * 16, 16)] + b_v[0, :]
        v1 = lg1_v[pl.ds(t * 16, 16)] + b_v[1, :]
        m = jnp.maximum(v0, v1)
        e = jnp.exp(-jnp.abs(v0 - v1))
        z = e / (2.0 + e)
        z2 = z * z
        logu = 2.0 * z * (1.0 + z2 * (1.0 / 3.0 + z2 * (0.2 + z2 * (1.0 / 7.0))))
        lse = m + logu
        o_v[0, pl.ds(t * 16, 16)] = v0 - lse
        o_v[1, pl.ds(t * 16, 16)] = v1 - lse

    base = wid * SPW
    pltpu.sync_copy(o_v.at[0], out_hbm.at[0, pl.ds(base, SPW)])
    pltpu.sync_copy(o_v.at[1], out_hbm.at[1, pl.ds(base, SPW)])


_sc_call = pl.kernel(
    _sc_body,
    out_type=jax.ShapeDtypeStruct((NCLS, B), jnp.float32),
    mesh=plsc.VectorSubcoreMesh(core_axis_name="c", subcore_axis_name="s",
                                num_cores=2, num_subcores=16),
    scratch_types=[
        pltpu.VMEM((CPW, CH), jnp.int32),
        pltpu.VMEM((NCLS, L * D), jnp.float32),
        pltpu.VMEM((NCLS, 16), jnp.float32),
        pltpu.VMEM((G * L, D), jnp.float32),
        pltpu.VMEM((G * L, D), jnp.float32),
        pltpu.VMEM((SPW,), jnp.float32),
        pltpu.VMEM((SPW,), jnp.float32),
        pltpu.VMEM((NCLS, SPW), jnp.float32),
        pltpu.SemaphoreType.DMA,
        pltpu.SemaphoreType.DMA,
    ],
    compiler_params=pltpu.CompilerParams(needs_layout_passes=False,
                                         use_tc_tiling_on_sc=False),
)


def kernel(input_data, emb_table, W, b):
    idx = input_data.astype(jnp.int32).reshape(B * L // CH, CH)
    wt = W.astype(jnp.float32).T
    b16 = jnp.broadcast_to(b.astype(jnp.float32).reshape(NCLS, 1), (NCLS, 16))
    outT = _sc_call(idx, emb_table.astype(jnp.float32), wt, b16)
    return outT.T


# speedup vs baseline: 6.9239x; 1.1728x over previous
import jax
import jax.numpy as jnp
from jax import lax
from jax.experimental import pallas as pl
from jax.experimental.pallas import tpu as pltpu
from jax.experimental.pallas import tpu_sc as plsc

V = 100000
D = 64
L = 200
B = 1024
NCLS = 2
CH = 80
NW = 32
SPW = B // NW
G = 4
NG = SPW // G
CPG = G * L // CH
IPW = SPW * L


def _sc_body(idx_hbm, tab_hbm, wt_hbm, b_hbm, out_hbm,
             idx_v, w_v, b_v, rows0, rows1, lg0_v, lg1_v, o_v, sem0, sem1):
    cid = lax.axis_index("c")
    sid = lax.axis_index("s")
    wid = sid * 2 + cid

    pltpu.sync_copy(idx_hbm.at[pl.ds(wid * IPW, IPW)], idx_v)
    pltpu.sync_copy(wt_hbm, w_v)
    pltpu.sync_copy(b_hbm, b_v)

    bufs = (rows0, rows1)
    sems = (sem0, sem1)

    def start_group(g, p):
        for k in range(CPG):
            pltpu.async_copy(
                tab_hbm.at[idx_v.at[pl.ds(g * (G * L) + k * CH, CH)]],
                bufs[p].at[pl.ds(k * CH, CH)], sems[p])

    def wait_group(p):
        for k in range(CPG):
            pltpu.make_async_copy(
                tab_hbm.at[idx_v.at[pl.ds(k * CH, CH)]],
                bufs[p].at[pl.ds(k * CH, CH)], sems[p]).wait()

    def compute_group(g, p):
        rows = bufs[p]

        def jbody(j, accs):
            off = j * 2 * D
            w = []
            for c in range(NCLS):
                for d in range(2):
                    wv = w_v[pl.ds(off + (c * 2 + d) * 32, 32)]
                    a, bb = plsc.unpack(wv, format=plsc.PackFormat.INTERLEAVED)
                    w.append(a)
                    w.append(bb)
            new = list(accs)
            for s in range(G):
                r = s * L + j
                for d in range(4):
                    f = rows[r, pl.ds(d * 16, 16)]
                    new[s * 2] = new[s * 2] + f * w[d]
                    new[s * 2 + 1] = new[s * 2 + 1] + f * w[4 + d]
            return tuple(new)

        z16 = jnp.zeros((16,), jnp.float32)
        accs = lax.fori_loop(0, L, jbody, (z16,) * (2 * G))
        lane0 = lax.iota(jnp.int32, 16) == 0
        for s in range(G):
            sg = g * G + s
            idxv = jnp.full((16,), sg, dtype=jnp.int32)
            s0 = jnp.full((16,), jnp.sum(accs[s * 2]))
            s1 = jnp.full((16,), jnp.sum(accs[s * 2 + 1]))
            plsc.store_scatter(lg0_v, [idxv], s0, mask=lane0)
            plsc.store_scatter(lg1_v, [idxv], s1, mask=lane0)

    start_group(0, 0)

    @pl.loop(0, NG, step=2)
    def _(g):
        start_group(g + 1, 1)
        wait_group(0)
        compute_group(g, 0)

        @pl.when(g + 2 < NG)
        def _():
            start_group(g + 2, 0)

        wait_group(1)
        compute_group(g + 1, 1)

    for t in range(SPW // 16):
        v0 = lg0_v[pl.ds(t * 16, 16)] + b_v[pl.ds(0, 16)]
        v1 = lg1_v[pl.ds(t * 16, 16)] + b_v[pl.ds(16, 16)]
        m = jnp.maximum(v0, v1)
        e = jnp.exp(-jnp.abs(v0 - v1))
        z = e / (2.0 + e)
        z2 = z * z
        logu = 2.0 * z * (1.0 + z2 * (1.0 / 3.0 + z2 * (0.2 + z2 * (1.0 / 7.0))))
        lse = m + logu
        o_v[0, pl.ds(t * 16, 16)] = v0 - lse
        o_v[1, pl.ds(t * 16, 16)] = v1 - lse

    base = wid * SPW
    pltpu.sync_copy(o_v.at[0], out_hbm.at[pl.ds(base, SPW)])
    pltpu.sync_copy(o_v.at[1], out_hbm.at[pl.ds(B + base, SPW)])


_sc_call = pl.kernel(
    _sc_body,
    out_type=jax.ShapeDtypeStruct((NCLS * B,), jnp.float32),
    mesh=plsc.VectorSubcoreMesh(core_axis_name="c", subcore_axis_name="s",
                                num_cores=2, num_subcores=16),
    scratch_types=[
        pltpu.VMEM((IPW,), jnp.int32),
        pltpu.VMEM((NCLS * L * D,), jnp.bfloat16),
        pltpu.VMEM((2 * 16,), jnp.float32),
        pltpu.VMEM((G * L, D), jnp.float32),
        pltpu.VMEM((G * L, D), jnp.float32),
        pltpu.VMEM((SPW,), jnp.float32),
        pltpu.VMEM((SPW,), jnp.float32),
        pltpu.VMEM((NCLS, SPW), jnp.float32),
        pltpu.SemaphoreType.DMA,
        pltpu.SemaphoreType.DMA,
    ],
    compiler_params=pltpu.CompilerParams(needs_layout_passes=False,
                                         use_tc_tiling_on_sc=False),
)


def kernel(input_data, emb_table, W, b):
    idx = input_data.astype(jnp.int32).reshape(B * L)
    wt = W.astype(jnp.float32).T.reshape(NCLS, L, 2, 2, 16)
    wt = wt.transpose(1, 0, 2, 4, 3)
    wbf = wt.astype(jnp.bfloat16).reshape(NCLS * L * D)
    b32 = jnp.repeat(b.astype(jnp.float32), 16)
    out = _sc_call(idx, emb_table.astype(jnp.float32), wbf, b32)
    return out.reshape(NCLS, B).T


# speedup vs baseline: 7.1052x; 1.0262x over previous
import jax
import jax.numpy as jnp
from jax import lax
from jax.experimental import pallas as pl
from jax.experimental.pallas import tpu as pltpu
from jax.experimental.pallas import tpu_sc as plsc

V = 100000
D = 64
L = 200
B = 1024
NCLS = 2
CH = 80
NW = 32
SPW = B // NW
G = 4
NG = SPW // G
CPG = G * L // CH
IPW = SPW * L


def _sc_body(idx_hbm, tab_hbm, wt_hbm, b_hbm, out_hbm,
             idx_v, w_v, b_v, rows0, rows1, lg0_v, lg1_v, o_v, sem0, sem1,
             semw):
    cid = lax.axis_index("c")
    sid = lax.axis_index("s")
    wid = sid * 2 + cid

    wcp = pltpu.async_copy(wt_hbm, w_v, semw)
    bcp = pltpu.async_copy(b_hbm, b_v, semw)
    pltpu.sync_copy(idx_hbm.at[pl.ds(wid * IPW, IPW)], idx_v)

    bufs = (rows0, rows1)
    sems = (sem0, sem1)

    def start_group(g, p):
        for k in range(CPG):
            pltpu.async_copy(
                tab_hbm.at[idx_v.at[pl.ds(g * (G * L) + k * CH, CH)]],
                bufs[p].at[pl.ds(k * CH, CH)], sems[p])

    def wait_group(p):
        for k in range(CPG):
            pltpu.make_async_copy(
                tab_hbm.at[idx_v.at[pl.ds(k * CH, CH)]],
                bufs[p].at[pl.ds(k * CH, CH)], sems[p]).wait()

    def compute_group(g, p):
        rows = bufs[p]

        def jbody(j, accs):
            off = j * 2 * D
            w0 = []
            w1 = []
            for d in range(4):
                wv = w_v[pl.ds(off + d * 32, 32)]
                a, bb = plsc.unpack(wv, format=plsc.PackFormat.INTERLEAVED)
                w0.append(a)
                w1.append(bb)
            new = list(accs)
            for s in range(G):
                r = s * L + j
                for d in range(4):
                    f = rows[r, pl.ds(d * 16, 16)]
                    new[s * 2] = new[s * 2] + f * w0[d]
                    new[s * 2 + 1] = new[s * 2 + 1] + f * w1[d]
            return tuple(new)

        z16 = jnp.zeros((16,), jnp.float32)
        accs = lax.fori_loop(0, L, jbody, (z16,) * (2 * G))
        lane0 = lax.iota(jnp.int32, 16) == 0
        for s in range(G):
            sg = g * G + s
            idxv = jnp.full((16,), sg, dtype=jnp.int32)
            s0 = jnp.full((16,), jnp.sum(accs[s * 2]))
            s1 = jnp.full((16,), jnp.sum(accs[s * 2 + 1]))
            plsc.store_scatter(lg0_v, [idxv], s0, mask=lane0)
            plsc.store_scatter(lg1_v, [idxv], s1, mask=lane0)

    start_group(0, 0)
    wcp.wait()
    bcp.wait()

    @pl.loop(0, NG, step=2)
    def _(g):
        start_group(g + 1, 1)
        wait_group(0)
        compute_group(g, 0)

        @pl.when(g + 2 < NG)
        def _():
            start_group(g + 2, 0)

        wait_group(1)
        compute_group(g + 1, 1)

    for t in range(SPW // 16):
        v0 = lg0_v[pl.ds(t * 16, 16)] + b_v[pl.ds(0, 16)]
        v1 = lg1_v[pl.ds(t * 16, 16)] + b_v[pl.ds(16, 16)]
        m = jnp.maximum(v0, v1)
        e = jnp.exp(-jnp.abs(v0 - v1))
        z = e / (2.0 + e)
        z2 = z * z
        logu = 2.0 * z * (1.0 + z2 * (1.0 / 3.0 + z2 * (0.2 + z2 * (1.0 / 7.0))))
        lse = m + logu
        o_v[0, pl.ds(t * 16, 16)] = v0 - lse
        o_v[1, pl.ds(t * 16, 16)] = v1 - lse

    base = wid * SPW
    pltpu.sync_copy(o_v.at[0], out_hbm.at[pl.ds(base, SPW)])
    pltpu.sync_copy(o_v.at[1], out_hbm.at[pl.ds(B + base, SPW)])


_sc_call = pl.kernel(
    _sc_body,
    out_type=jax.ShapeDtypeStruct((NCLS * B,), jnp.float32),
    mesh=plsc.VectorSubcoreMesh(core_axis_name="c", subcore_axis_name="s",
                                num_cores=2, num_subcores=16),
    scratch_types=[
        pltpu.VMEM((IPW,), jnp.int32),
        pltpu.VMEM((NCLS * L * D,), jnp.bfloat16),
        pltpu.VMEM((2 * 16,), jnp.float32),
        pltpu.VMEM((G * L, D), jnp.float32),
        pltpu.VMEM((G * L, D), jnp.float32),
        pltpu.VMEM((SPW,), jnp.float32),
        pltpu.VMEM((SPW,), jnp.float32),
        pltpu.VMEM((NCLS, SPW), jnp.float32),
        pltpu.SemaphoreType.DMA,
        pltpu.SemaphoreType.DMA,
        pltpu.SemaphoreType.DMA,
    ],
    compiler_params=pltpu.CompilerParams(needs_layout_passes=False,
                                         use_tc_tiling_on_sc=False),
)


def kernel(input_data, emb_table, W, b):
    idx = input_data.astype(jnp.int32).reshape(B * L)
    wbf = W.astype(jnp.bfloat16).reshape(NCLS * L * D)
    b32 = jnp.repeat(b.astype(jnp.float32), 16)
    out = _sc_call(idx, emb_table.astype(jnp.float32), wbf, b32)
    return out.reshape(NCLS, B).T


# speedup vs baseline: 7.1249x; 1.0028x over previous
import jax
import jax.numpy as jnp
from jax import lax
from jax.experimental import pallas as pl
from jax.experimental.pallas import tpu as pltpu
from jax.experimental.pallas import tpu_sc as plsc

V = 100000
D = 64
L = 200
B = 1024
NCLS = 2
NW = 32
SPW = B // NW
G = 4
NG = SPW // G


def _sc_body(idx_hbm, tab_hbm, wt_hbm, b_hbm, out_hbm,
             idx_v, w_v, b_v, rows0, rows1, lg0_v, lg1_v, o_v, sem0, sem1,
             semw):
    cid = lax.axis_index("c")
    sid = lax.axis_index("s")
    wid = sid * 2 + cid

    wcp = pltpu.async_copy(wt_hbm, w_v, semw)
    bcp = pltpu.async_copy(b_hbm, b_v, semw)
    pltpu.sync_copy(idx_hbm.at[pl.ds(wid * SPW, SPW)], idx_v)

    bufs = (rows0, rows1)
    sems = (sem0, sem1)

    def start_group(g, p):
        for s in range(G):
            for c, (off, n) in enumerate(((0, 128), (128, L - 128))):
                pltpu.async_copy(
                    tab_hbm.at[idx_v.at[g * G + s, pl.ds(off, n)]],
                    bufs[p].at[pl.ds(s * L + off, n)], sems[p])

    def wait_group(p):
        for s in range(G):
            for c, (off, n) in enumerate(((0, 128), (128, L - 128))):
                pltpu.make_async_copy(
                    tab_hbm.at[idx_v.at[s, pl.ds(off, n)]],
                    bufs[p].at[pl.ds(s * L + off, n)], sems[p]).wait()

    def compute_group(g, p):
        rows = bufs[p]

        def jbody(j, accs):
            off = j * 2 * D
            w0 = []
            w1 = []
            for d in range(4):
                wv = w_v[pl.ds(off + d * 32, 32)]
                a, bb = plsc.unpack(wv, format=plsc.PackFormat.INTERLEAVED)
                w0.append(a)
                w1.append(bb)
            new = list(accs)
            for s in range(G):
                r = s * L + j
                for d in range(4):
                    f = rows[r, pl.ds(d * 16, 16)]
                    new[s * 2] = new[s * 2] + f * w0[d]
                    new[s * 2 + 1] = new[s * 2 + 1] + f * w1[d]
            return tuple(new)

        z16 = jnp.zeros((16,), jnp.float32)
        accs = lax.fori_loop(0, L, jbody, (z16,) * (2 * G))
        lane0 = lax.iota(jnp.int32, 16) == 0
        for s in range(G):
            sg = g * G + s
            idxv = jnp.full((16,), sg, dtype=jnp.int32)
            s0 = jnp.full((16,), jnp.sum(accs[s * 2]))
            s1 = jnp.full((16,), jnp.sum(accs[s * 2 + 1]))
            plsc.store_scatter(lg0_v, [idxv], s0, mask=lane0)
            plsc.store_scatter(lg1_v, [idxv], s1, mask=lane0)

    start_group(0, 0)
    wcp.wait()
    bcp.wait()

    @pl.loop(0, NG, step=2)
    def _(g):
        start_group(g + 1, 1)
        wait_group(0)
        compute_group(g, 0)

        @pl.when(g + 2 < NG)
        def _():
            start_group(g + 2, 0)

        wait_group(1)
        compute_group(g + 1, 1)

    for t in range(SPW // 16):
        v0 = lg0_v[pl.ds(t * 16, 16)] + b_v[pl.ds(0, 16)]
        v1 = lg1_v[pl.ds(t * 16, 16)] + b_v[pl.ds(16, 16)]
        m = jnp.maximum(v0, v1)
        e = jnp.exp(-jnp.abs(v0 - v1))
        z = e / (2.0 + e)
        z2 = z * z
        logu = 2.0 * z * (1.0 + z2 * (1.0 / 3.0 + z2 * (0.2 + z2 * (1.0 / 7.0))))
        lse = m + logu
        o_v[0, pl.ds(t * 16, 16)] = v0 - lse
        o_v[1, pl.ds(t * 16, 16)] = v1 - lse

    base = wid * SPW
    pltpu.sync_copy(o_v.at[0], out_hbm.at[pl.ds(base, SPW)])
    pltpu.sync_copy(o_v.at[1], out_hbm.at[pl.ds(B + base, SPW)])


_sc_call = pl.kernel(
    _sc_body,
    out_type=jax.ShapeDtypeStruct((NCLS * B,), jnp.float32),
    mesh=plsc.VectorSubcoreMesh(core_axis_name="c", subcore_axis_name="s",
                                num_cores=2, num_subcores=16),
    scratch_types=[
        pltpu.VMEM((SPW, L), jnp.int32),
        pltpu.VMEM((NCLS * L * D,), jnp.bfloat16),
        pltpu.VMEM((2 * 16,), jnp.float32),
        pltpu.VMEM((G * L, D), jnp.float32),
        pltpu.VMEM((G * L, D), jnp.float32),
        pltpu.VMEM((SPW,), jnp.float32),
        pltpu.VMEM((SPW,), jnp.float32),
        pltpu.VMEM((NCLS, SPW), jnp.float32),
        pltpu.SemaphoreType.DMA,
        pltpu.SemaphoreType.DMA,
        pltpu.SemaphoreType.DMA,
    ],
    compiler_params=pltpu.CompilerParams(needs_layout_passes=False,
                                         use_tc_tiling_on_sc=False),
)


def kernel(input_data, emb_table, W, b):
    idx = input_data.astype(jnp.int32)
    wbf = W.astype(jnp.bfloat16).reshape(NCLS * L * D)
    b32 = jnp.repeat(b.astype(jnp.float32), 16)
    out = _sc_call(idx, emb_table.astype(jnp.float32), wbf, b32)
    return out.reshape(NCLS, B).T


# speedup vs baseline: 7.1309x; 1.0008x over previous
import jax
import jax.numpy as jnp
from jax import lax
from jax.experimental import pallas as pl
from jax.experimental.pallas import tpu as pltpu
from jax.experimental.pallas import tpu_sc as plsc

V = 100000
D = 64
L = 200
B = 1024
NCLS = 2
NW = 32
SPW = B // NW
G = 4
NG = SPW // G


def _sc_body(idx_hbm, tab_hbm, wt_hbm, b_hbm, out_hbm,
             idx_v, w_v, b_v, rows0, rows1, lg0_v, lg1_v, o_v, sem0, sem1,
             semw):
    cid = lax.axis_index("c")
    sid = lax.axis_index("s")
    wid = sid * 2 + cid

    wcp = pltpu.async_copy(wt_hbm, w_v, semw)
    bcp = pltpu.async_copy(b_hbm, b_v, semw)
    pltpu.sync_copy(idx_hbm.at[pl.ds(wid * SPW, SPW)], idx_v)

    bufs = (rows0, rows1)
    sems = (sem0, sem1)

    def start_group(g, p):
        for s in range(G):
            for c, (off, n) in enumerate(((0, 128), (128, L - 128))):
                pltpu.async_copy(
                    tab_hbm.at[idx_v.at[g * G + s, pl.ds(off, n)]],
                    bufs[p].at[pl.ds(s * L + off, n)], sems[p])

    def wait_group(p):
        for s in range(G):
            for c, (off, n) in enumerate(((0, 128), (128, L - 128))):
                pltpu.make_async_copy(
                    tab_hbm.at[idx_v.at[s, pl.ds(off, n)]],
                    bufs[p].at[pl.ds(s * L + off, n)], sems[p]).wait()

    def compute_group(g, p):
        rows = bufs[p]

        def jbody(j, accs):
            off = j * 2 * D
            w0 = []
            w1 = []
            for d in range(4):
                wv = w_v[pl.ds(off + d * 32, 32)]
                a, bb = plsc.unpack(wv, format=plsc.PackFormat.INTERLEAVED)
                w0.append(a)
                w1.append(bb)
            new = list(accs)
            for s in range(G):
                r = s * L + j
                for d in range(4):
                    f = rows[r, pl.ds(d * 16, 16)]
                    new[s * 2] = new[s * 2] + f * w0[d]
                    new[s * 2 + 1] = new[s * 2 + 1] + f * w1[d]
            return tuple(new)

        z16 = jnp.zeros((16,), jnp.float32)
        accs = lax.fori_loop(0, L, jbody, (z16,) * (2 * G))
        lane0 = lax.iota(jnp.int32, 16) == 0
        for s in range(G):
            sg = g * G + s
            idxv = jnp.full((16,), sg, dtype=jnp.int32)
            s0 = jnp.full((16,), jnp.sum(accs[s * 2]))
            s1 = jnp.full((16,), jnp.sum(accs[s * 2 + 1]))
            plsc.store_scatter(lg0_v, [idxv], s0, mask=lane0)
            plsc.store_scatter(lg1_v, [idxv], s1, mask=lane0)

    start_group(0, 0)
    wcp.wait()
    bcp.wait()

    @pl.loop(0, NG, step=2)
    def _(g):
        start_group(g + 1, 1)
        wait_group(0)
        compute_group(g, 0)

        @pl.when(g + 2 < NG)
        def _():
            start_group(g + 2, 0)

        wait_group(1)
        compute_group(g + 1, 1)

    for t in range(SPW // 16):
        v0 = lg0_v[pl.ds(t * 16, 16)] + b_v[pl.ds(0, 16)]
        v1 = lg1_v[pl.ds(t * 16, 16)] + b_v[pl.ds(16, 16)]
        m = jnp.maximum(v0, v1)
        e = jnp.exp(-jnp.abs(v0 - v1))
        z = e / (2.0 + e)
        z2 = z * z
        logu = 2.0 * z * (1.0 + z2 * (1.0 / 3.0 + z2 * (0.2 + z2 * (1.0 / 7.0))))
        lse = m + logu
        o_v[0, pl.ds(t * 16, 16)] = v0 - lse
        o_v[1, pl.ds(t * 16, 16)] = v1 - lse

    base = wid * SPW
    pltpu.sync_copy(o_v.at[0], out_hbm.at[pl.ds(base, SPW)])
    pltpu.sync_copy(o_v.at[1], out_hbm.at[pl.ds(B + base, SPW)])


_sc_call = pl.kernel(
    _sc_body,
    out_type=jax.ShapeDtypeStruct((NCLS * B,), jnp.float32),
    mesh=plsc.VectorSubcoreMesh(core_axis_name="c", subcore_axis_name="s",
                                num_cores=2, num_subcores=16),
    scratch_types=[
        pltpu.VMEM((SPW, L), jnp.int32),
        pltpu.VMEM((NCLS * L * D,), jnp.bfloat16),
        pltpu.VMEM((2 * 16,), jnp.float32),
        pltpu.VMEM((G * L, D), jnp.float32),
        pltpu.VMEM((G * L, D), jnp.float32),
        pltpu.VMEM((SPW,), jnp.float32),
        pltpu.VMEM((SPW,), jnp.float32),
        pltpu.VMEM((NCLS, SPW), jnp.float32),
        pltpu.SemaphoreType.DMA,
        pltpu.SemaphoreType.DMA,
        pltpu.SemaphoreType.DMA,
    ],
    compiler_params=pltpu.CompilerParams(needs_layout_passes=False,
                                         use_tc_tiling_on_sc=False),
)


def kernel(input_data, emb_table, W, b):
    idx = input_data.astype(jnp.int32)
    wbf = W.reshape(NCLS * L * D).astype(jnp.bfloat16)
    b32 = jnp.repeat(b.astype(jnp.float32), 16)
    out = _sc_call(idx, emb_table.astype(jnp.float32), wbf, b32)
    return out.reshape(NCLS, B).T
